# baseline (speedup 1.0000x reference)
import functools

import jax
import jax.numpy as jnp
from jax import lax
from jax.experimental import pallas as pl
from jax.experimental.pallas import tpu as pltpu
from jax.experimental.pallas import tpu_sc as plsc

B, S, H = 1, 2048, 1024
E, K, F = 16, 2, 2048
T = B * S
BLK = 128
NB = (T * K) // BLK + (E - 1)
P = NB * BLK

NC, NS = 2, 16
NW = NC * NS
TPW = T // NW
SUB = 32
FBLK = 512
NF = F // FBLK


def _gelu_exact(x):
    return 0.5 * x * (1.0 + lax.erf(x * 0.7071067811865476))


def _dispatch_body(x_ref, wr_ref, pos0_ref, pos1_ref, w0_ref, w1_ref,
                   be_ref, na_ref):
    x = x_ref[...]
    wr = wr_ref[...]
    logits = lax.dot_general(x, wr, (((1,), (1,)), ((), ())),
                             preferred_element_type=jnp.float32)
    eidx = lax.broadcasted_iota(jnp.int32, (T, E), 1)
    m1 = jnp.max(logits, axis=1, keepdims=True)
    idx1 = jnp.min(jnp.where(logits == m1, eidx, E), axis=1, keepdims=True)
    oh1 = eidx == idx1
    logits2 = jnp.where(oh1, -jnp.inf, logits)
    m2 = jnp.max(logits2, axis=1, keepdims=True)
    idx2 = jnp.min(jnp.where(logits2 == m2, eidx, E), axis=1, keepdims=True)
    oh2 = eidx == idx2
    e2 = jnp.exp(m2 - m1)
    wt1 = 1.0 / (1.0 + e2)
    wt2 = 1.0 - wt1

    oh = (oh1 | oh2).astype(jnp.float32)
    r_i = lax.broadcasted_iota(jnp.int32, (T, T), 0)
    c_i = lax.broadcasted_iota(jnp.int32, (T, T), 1)
    ltri = (c_i < r_i).astype(jnp.bfloat16)
    cum_excl = lax.dot_general(ltri, oh.astype(jnp.bfloat16),
                               (((1,), (0,)), ((), ())),
                               preferred_element_type=jnp.float32)
    counts = jnp.sum(oh, axis=0, keepdims=True)
    counts_i = counts.astype(jnp.int32)
    padded = ((counts_i + BLK - 1) // BLK) * BLK
    e_r = lax.broadcasted_iota(jnp.int32, (E, E), 0)
    e_c = lax.broadcasted_iota(jnp.int32, (E, E), 1)
    ue = (e_r < e_c).astype(jnp.float32)
    off_f = lax.dot_general(padded.astype(jnp.float32), ue,
                            (((1,), (0,)), ((), ())),
                            preferred_element_type=jnp.float32)
    rank0 = jnp.sum(jnp.where(oh1, cum_excl, 0.0), axis=1, keepdims=True)
    rank1 = jnp.sum(jnp.where(oh2, cum_excl, 0.0), axis=1, keepdims=True)
    off0 = jnp.sum(jnp.where(oh1, off_f, 0.0), axis=1, keepdims=True)
    off1 = jnp.sum(jnp.where(oh2, off_f, 0.0), axis=1, keepdims=True)
    pos0_ref[...] = (rank0 + off0).astype(jnp.int32)
    pos1_ref[...] = (rank1 + off1).astype(jnp.int32)
    w0_ref[...] = jnp.broadcast_to(wt1, (T, 128))
    w1_ref[...] = jnp.broadcast_to(wt2, (T, 128))

    off_i = off_f.astype(jnp.int32)
    endblk = (off_i + padded) // BLK
    total = jnp.sum(padded)
    nact = total // BLK
    b_i = lax.broadcasted_iota(jnp.int32, (NB, E), 0)
    be = jnp.sum((jnp.broadcast_to(endblk, (NB, E)) <= b_i).astype(jnp.int32),
                 axis=1, keepdims=True)
    be_last = jnp.sum((endblk <= nact - 1).astype(jnp.int32))
    b_col = lax.broadcasted_iota(jnp.int32, (NB, 1), 0)
    be_ref[...] = jnp.where(b_col >= nact, be_last, be)
    na_ref[...] = jnp.full((1, 1), nact, jnp.int32)


_DISPATCH_OUT = [
    jax.ShapeDtypeStruct((T, 1), jnp.int32),
    jax.ShapeDtypeStruct((T, 1), jnp.int32),
    jax.ShapeDtypeStruct((T, 128), jnp.float32),
    jax.ShapeDtypeStruct((T, 128), jnp.float32),
    jax.ShapeDtypeStruct((NB, 1), jnp.int32),
    jax.ShapeDtypeStruct((1, 1), jnp.int32),
]


@functools.lru_cache(maxsize=1)
def _scatter_sc():
    mesh = plsc.VectorSubcoreMesh(core_axis_name="c", subcore_axis_name="s",
                                  num_cores=NC, num_subcores=NS)

    @functools.partial(
        pl.kernel,
        out_type=[jax.ShapeDtypeStruct((P, H), jnp.float32),
                  jax.ShapeDtypeStruct((P, 128), jnp.float32)],
        mesh=mesh,
        scratch_types=[pltpu.VMEM((TPW, H), jnp.float32),
                       pltpu.VMEM((TPW,), jnp.int32),
                       pltpu.VMEM((TPW, 128), jnp.float32),
                       pltpu.SemaphoreType.DMA,
                       pltpu.SemaphoreType.DMA],
    )
    def scatter(x_hbm, pos0_hbm, pos1_hbm, w0_hbm, w1_hbm,
                xg_hbm, ws_hbm, rows_v, idx_v, w_v, sem, sem2):
        wid = lax.axis_index("s") * NC + lax.axis_index("c")
        base = wid * TPW
        pltpu.sync_copy(x_hbm.at[pl.ds(base, TPW)], rows_v)
        for p_hbm, wk_hbm in ((pos0_hbm, w0_hbm), (pos1_hbm, w1_hbm)):
            pltpu.sync_copy(p_hbm.at[pl.ds(base, TPW)], idx_v)
            pltpu.sync_copy(wk_hbm.at[pl.ds(base, TPW)], w_v)
            c0 = pltpu.async_copy(rows_v, xg_hbm.at[idx_v], sem)
            c1 = pltpu.async_copy(w_v, ws_hbm.at[idx_v], sem2)
            c0.wait()
            c1.wait()

    return scatter


def _mlp_body(be_ref, na_ref, xg_ref, ws_ref, w1_ref, b1_ref, w2_ref, b2_ref,
              y_ref, xb_scr):
    b = pl.program_id(0)
    f = pl.program_id(1)

    @pl.when(b < na_ref[0])
    def _():
        @pl.when(f == 0)
        def _():
            xb_scr[...] = xg_ref[...].astype(jnp.bfloat16)
            y_ref[...] = jnp.zeros((BLK, H), jnp.float32)

        h = lax.dot_general(xb_scr[...], w1_ref[0].astype(jnp.bfloat16),
                            (((1,), (0,)), ((), ())),
                            preferred_element_type=jnp.float32)
        h = _gelu_exact(h + b1_ref[0]).astype(jnp.bfloat16)
        y_ref[...] += lax.dot_general(h, w2_ref[0].astype(jnp.bfloat16),
                                      (((1,), (0,)), ((), ())),
                                      preferred_element_type=jnp.float32)

        @pl.when(f == NF - 1)
        def _():
            y_ref[...] = (y_ref[...] + b2_ref[0]) * ws_ref[...][:, 0:1]


def _mlp_grid_spec():
    return pltpu.PrefetchScalarGridSpec(
        num_scalar_prefetch=2,
        grid=(NB, NF),
        in_specs=[
            pl.BlockSpec((BLK, H), lambda b, f, be, na: (b, 0)),
            pl.BlockSpec((BLK, 128), lambda b, f, be, na: (b, 0)),
            pl.BlockSpec((1, H, FBLK), lambda b, f, be, na: (be[b], 0, f)),
            pl.BlockSpec((1, 1, FBLK), lambda b, f, be, na: (be[b], 0, f)),
            pl.BlockSpec((1, FBLK, H), lambda b, f, be, na: (be[b], f, 0)),
            pl.BlockSpec((1, 1, H), lambda b, f, be, na: (be[b], 0, 0)),
        ],
        out_specs=pl.BlockSpec((BLK, H), lambda b, f, be, na: (b, 0)),
        scratch_shapes=[pltpu.VMEM((BLK, H), jnp.bfloat16)],
    )


@functools.lru_cache(maxsize=1)
def _combine_sc():
    mesh = plsc.VectorSubcoreMesh(core_axis_name="c", subcore_axis_name="s",
                                  num_cores=NC, num_subcores=NS)

    @functools.partial(
        pl.kernel,
        out_type=jax.ShapeDtypeStruct((T, H), jnp.float32),
        mesh=mesh,
        scratch_types=[pltpu.VMEM((SUB, H), jnp.float32),
                       pltpu.VMEM((SUB, H), jnp.float32),
                       pltpu.VMEM((SUB,), jnp.int32),
                       pltpu.VMEM((SUB,), jnp.int32),
                       pltpu.SemaphoreType.DMA,
                       pltpu.SemaphoreType.DMA],
    )
    def combine(y_hbm, pos0_hbm, pos1_hbm, out_hbm,
                buf0, buf1, i0, i1, s0, s1):
        wid = lax.axis_index("s") * NC + lax.axis_index("c")
        for sub in range(TPW // SUB):
            base = wid * TPW + sub * SUB
            pltpu.sync_copy(pos0_hbm.at[pl.ds(base, SUB)], i0)
            pltpu.sync_copy(pos1_hbm.at[pl.ds(base, SUB)], i1)
            c0 = pltpu.async_copy(y_hbm.at[i0], buf0, s0)
            c1 = pltpu.async_copy(y_hbm.at[i1], buf1, s1)
            c0.wait()
            c1.wait()

            def row_body(r, _):
                def col_body(c, _):
                    sl = pl.ds(c * 16, 16)
                    buf0[r, sl] = buf0[r, sl] + buf1[r, sl]
                    return 0
                return lax.fori_loop(0, H // 16, col_body, 0)

            lax.fori_loop(0, SUB, row_body, 0)
            pltpu.sync_copy(buf0, out_hbm.at[pl.ds(base, SUB)])

    return combine


def kernel(hidden_states, W_router, W1, b1, W2, b2):
    x2d = hidden_states.reshape(T, H)
    pos0, pos1, w0r, w1r, be, na = pl.pallas_call(
        _dispatch_body, out_shape=_DISPATCH_OUT)(x2d, W_router)
    pos0 = pos0.reshape(T)
    pos1 = pos1.reshape(T)

    xg, ws = _scatter_sc()(x2d, pos0, pos1, w0r, w1r)

    y = pl.pallas_call(
        _mlp_body,
        grid_spec=_mlp_grid_spec(),
        out_shape=jax.ShapeDtypeStruct((P, H), jnp.float32),
    )(be.reshape(NB), na.reshape(1), xg, ws,
      W1, b1.reshape(E, 1, F), W2, b2.reshape(E, 1, H))

    out = _combine_sc()(y, pos0, pos1)
    return out.reshape(hidden_states.shape)


# speedup vs baseline: 1.9146x; 1.9146x over previous
import functools

import jax
import jax.numpy as jnp
from jax import lax
from jax.experimental import pallas as pl
from jax.experimental.pallas import tpu as pltpu
from jax.experimental.pallas import tpu_sc as plsc

B, S, H = 1, 2048, 1024
E, K, F = 16, 2, 2048
T = B * S
BLK = 256
NB = (T * K) // BLK + (E - 1)
P = NB * BLK

NC, NS = 2, 16
NW = NC * NS
TPW = T // NW
SUB = 32
FBLK = 512
NF = F // FBLK


def _gelu_exact(x):
    return 0.5 * x * (1.0 + lax.erf(x * 0.7071067811865476))


def _dispatch_body(x_ref, wr_ref, pos0_ref, pos1_ref, w0_ref, w1_ref,
                   be_ref, na_ref):
    x = x_ref[...]
    wr = wr_ref[...]
    logits = lax.dot_general(x, wr, (((1,), (1,)), ((), ())),
                             preferred_element_type=jnp.float32)
    eidx = lax.broadcasted_iota(jnp.int32, (T, E), 1)
    m1 = jnp.max(logits, axis=1, keepdims=True)
    idx1 = jnp.min(jnp.where(logits == m1, eidx, E), axis=1, keepdims=True)
    oh1 = eidx == idx1
    logits2 = jnp.where(oh1, -jnp.inf, logits)
    m2 = jnp.max(logits2, axis=1, keepdims=True)
    idx2 = jnp.min(jnp.where(logits2 == m2, eidx, E), axis=1, keepdims=True)
    oh2 = eidx == idx2
    e2 = jnp.exp(m2 - m1)
    wt1 = 1.0 / (1.0 + e2)
    wt2 = 1.0 - wt1

    oh = (oh1 | oh2).astype(jnp.float32)
    r_i = lax.broadcasted_iota(jnp.int32, (T, T), 0)
    c_i = lax.broadcasted_iota(jnp.int32, (T, T), 1)
    ltri = (c_i < r_i).astype(jnp.bfloat16)
    cum_excl = lax.dot_general(ltri, oh.astype(jnp.bfloat16),
                               (((1,), (0,)), ((), ())),
                               preferred_element_type=jnp.float32)
    counts = jnp.sum(oh, axis=0, keepdims=True)
    counts_i = counts.astype(jnp.int32)
    padded = ((counts_i + BLK - 1) // BLK) * BLK
    e_r = lax.broadcasted_iota(jnp.int32, (E, E), 0)
    e_c = lax.broadcasted_iota(jnp.int32, (E, E), 1)
    ue = (e_r < e_c).astype(jnp.float32)
    off_f = lax.dot_general(padded.astype(jnp.float32), ue,
                            (((1,), (0,)), ((), ())),
                            preferred_element_type=jnp.float32)
    rank0 = jnp.sum(jnp.where(oh1, cum_excl, 0.0), axis=1, keepdims=True)
    rank1 = jnp.sum(jnp.where(oh2, cum_excl, 0.0), axis=1, keepdims=True)
    off0 = jnp.sum(jnp.where(oh1, off_f, 0.0), axis=1, keepdims=True)
    off1 = jnp.sum(jnp.where(oh2, off_f, 0.0), axis=1, keepdims=True)
    pos0_ref[...] = (rank0 + off0).astype(jnp.int32)
    pos1_ref[...] = (rank1 + off1).astype(jnp.int32)
    w0_ref[...] = jnp.broadcast_to(wt1, (T, 128))
    w1_ref[...] = jnp.broadcast_to(wt2, (T, 128))

    off_i = off_f.astype(jnp.int32)
    endblk = (off_i + padded) // BLK
    total = jnp.sum(padded)
    nact = total // BLK
    b_i = lax.broadcasted_iota(jnp.int32, (NB, E), 0)
    be = jnp.sum((jnp.broadcast_to(endblk, (NB, E)) <= b_i).astype(jnp.int32),
                 axis=1, keepdims=True)
    be_last = jnp.sum((endblk <= nact - 1).astype(jnp.int32))
    b_col = lax.broadcasted_iota(jnp.int32, (NB, 1), 0)
    be_ref[...] = jnp.where(b_col >= nact, be_last, be)
    na_ref[...] = jnp.full((1, 1), nact, jnp.int32)


_DISPATCH_OUT = [
    jax.ShapeDtypeStruct((T, 1), jnp.int32),
    jax.ShapeDtypeStruct((T, 1), jnp.int32),
    jax.ShapeDtypeStruct((T, 128), jnp.float32),
    jax.ShapeDtypeStruct((T, 128), jnp.float32),
    jax.ShapeDtypeStruct((NB, 1), jnp.int32),
    jax.ShapeDtypeStruct((1, 1), jnp.int32),
]


@functools.lru_cache(maxsize=1)
def _scatter_sc():
    mesh = plsc.VectorSubcoreMesh(core_axis_name="c", subcore_axis_name="s",
                                  num_cores=NC, num_subcores=NS)

    @functools.partial(
        pl.kernel,
        out_type=[jax.ShapeDtypeStruct((P, H), jnp.float32),
                  jax.ShapeDtypeStruct((P, 128), jnp.float32)],
        mesh=mesh,
        scratch_types=[pltpu.VMEM((TPW, H), jnp.float32),
                       pltpu.VMEM((TPW,), jnp.int32),
                       pltpu.VMEM((TPW, 128), jnp.float32),
                       pltpu.SemaphoreType.DMA,
                       pltpu.SemaphoreType.DMA],
    )
    def scatter(x_hbm, pos0_hbm, pos1_hbm, w0_hbm, w1_hbm,
                xg_hbm, ws_hbm, rows_v, idx_v, w_v, sem, sem2):
        wid = lax.axis_index("s") * NC + lax.axis_index("c")
        base = wid * TPW
        pltpu.sync_copy(x_hbm.at[pl.ds(base, TPW)], rows_v)
        for p_hbm, wk_hbm in ((pos0_hbm, w0_hbm), (pos1_hbm, w1_hbm)):
            pltpu.sync_copy(p_hbm.at[pl.ds(base, TPW)], idx_v)
            pltpu.sync_copy(wk_hbm.at[pl.ds(base, TPW)], w_v)
            c0 = pltpu.async_copy(rows_v, xg_hbm.at[idx_v], sem)
            c1 = pltpu.async_copy(w_v, ws_hbm.at[idx_v], sem2)
            c0.wait()
            c1.wait()

    return scatter


def _mlp_body(be_ref, na_ref, xg_ref, ws_ref, w1_ref, b1_ref, w2_ref, b2_ref,
              y_ref, h_scr):
    s = pl.program_id(0)

    @pl.when(s < na_ref[0])
    def _():
        xb = xg_ref[...].astype(jnp.bfloat16)
        h = lax.dot_general(xb, w1_ref[0].astype(jnp.bfloat16),
                            (((1,), (0,)), ((), ())),
                            preferred_element_type=jnp.float32)
        h = _gelu_exact(h + b1_ref[0]).astype(jnp.bfloat16)
        h_scr[lax.rem(s, 2)] = h

    @pl.when(jnp.logical_and(s >= 1, s - 1 < na_ref[0]))
    def _():
        y = lax.dot_general(h_scr[lax.rem(s - 1, 2)],
                            w2_ref[0].astype(jnp.bfloat16),
                            (((1,), (0,)), ((), ())),
                            preferred_element_type=jnp.float32)
        y = y + b2_ref[0]
        y_ref[...] = y * ws_ref[...][:, 0:1]


def _mlp_grid_spec():
    c0 = lambda s: jnp.minimum(s, NB - 1)
    c1 = lambda s: jnp.maximum(s - 1, 0)
    return pltpu.PrefetchScalarGridSpec(
        num_scalar_prefetch=2,
        grid=(NB + 1,),
        in_specs=[
            pl.BlockSpec((BLK, H), lambda s, be, na: (c0(s), 0)),
            pl.BlockSpec((BLK, 128), lambda s, be, na: (c1(s), 0)),
            pl.BlockSpec((1, H, F), lambda s, be, na: (be[c0(s)], 0, 0)),
            pl.BlockSpec((1, 1, F), lambda s, be, na: (be[c0(s)], 0, 0)),
            pl.BlockSpec((1, F, H), lambda s, be, na: (be[c1(s)], 0, 0)),
            pl.BlockSpec((1, 1, H), lambda s, be, na: (be[c1(s)], 0, 0)),
        ],
        out_specs=pl.BlockSpec((BLK, H), lambda s, be, na: (c1(s), 0)),
        scratch_shapes=[pltpu.VMEM((2, BLK, F), jnp.bfloat16)],
    )


@functools.lru_cache(maxsize=1)
def _combine_sc():
    mesh = plsc.VectorSubcoreMesh(core_axis_name="c", subcore_axis_name="s",
                                  num_cores=NC, num_subcores=NS)

    @functools.partial(
        pl.kernel,
        out_type=jax.ShapeDtypeStruct((T, H), jnp.float32),
        mesh=mesh,
        scratch_types=[pltpu.VMEM((SUB, H), jnp.float32),
                       pltpu.VMEM((SUB, H), jnp.float32),
                       pltpu.VMEM((SUB,), jnp.int32),
                       pltpu.VMEM((SUB,), jnp.int32),
                       pltpu.SemaphoreType.DMA,
                       pltpu.SemaphoreType.DMA],
    )
    def combine(y_hbm, pos0_hbm, pos1_hbm, out_hbm,
                buf0, buf1, i0, i1, s0, s1):
        wid = lax.axis_index("s") * NC + lax.axis_index("c")
        for sub in range(TPW // SUB):
            base = wid * TPW + sub * SUB
            pltpu.sync_copy(pos0_hbm.at[pl.ds(base, SUB)], i0)
            pltpu.sync_copy(pos1_hbm.at[pl.ds(base, SUB)], i1)
            c0 = pltpu.async_copy(y_hbm.at[i0], buf0, s0)
            c1 = pltpu.async_copy(y_hbm.at[i1], buf1, s1)
            c0.wait()
            c1.wait()

            def row_body(r, _):
                def col_body(c, _):
                    sl = pl.ds(c * 16, 16)
                    buf0[r, sl] = buf0[r, sl] + buf1[r, sl]
                    return 0
                return lax.fori_loop(0, H // 16, col_body, 0)

            lax.fori_loop(0, SUB, row_body, 0)
            pltpu.sync_copy(buf0, out_hbm.at[pl.ds(base, SUB)])

    return combine


def kernel(hidden_states, W_router, W1, b1, W2, b2):
    x2d = hidden_states.reshape(T, H)
    pos0, pos1, w0r, w1r, be, na = pl.pallas_call(
        _dispatch_body, out_shape=_DISPATCH_OUT)(x2d, W_router)
    pos0 = pos0.reshape(T)
    pos1 = pos1.reshape(T)

    xg, ws = _scatter_sc()(x2d, pos0, pos1, w0r, w1r)

    y = pl.pallas_call(
        _mlp_body,
        grid_spec=_mlp_grid_spec(),
        out_shape=jax.ShapeDtypeStruct((P, H), jnp.float32),
    )(be.reshape(NB), na.reshape(1), xg, ws,
      W1, b1.reshape(E, 1, F), W2, b2.reshape(E, 1, H))

    out = _combine_sc()(y, pos0, pos1)
    return out.reshape(hidden_states.shape)


# speedup vs baseline: 2.0099x; 1.0498x over previous
import functools

import jax
import jax.numpy as jnp
from jax import lax
from jax.experimental import pallas as pl
from jax.experimental.pallas import tpu as pltpu
from jax.experimental.pallas import tpu_sc as plsc

B, S, H = 1, 2048, 1024
E, K, F = 16, 2, 2048
T = B * S
BLK = 256
NB = (T * K) // BLK + (E - 1)
P = NB * BLK

NC, NS = 2, 16
NW = NC * NS
TPW = T // NW
SUB = 32
FBLK = 512
NF = F // FBLK


def _gelu_exact(x):
    return 0.5 * x * (1.0 + lax.erf(x * 0.7071067811865476))


def _dispatch_body(x_ref, wr_ref, pos0_ref, pos1_ref, w0_ref, w1_ref,
                   be_ref, na_ref):
    x = x_ref[...]
    wr = wr_ref[...]
    logits = lax.dot_general(x, wr, (((1,), (1,)), ((), ())),
                             preferred_element_type=jnp.float32)
    eidx = lax.broadcasted_iota(jnp.int32, (T, E), 1)
    m1 = jnp.max(logits, axis=1, keepdims=True)
    idx1 = jnp.min(jnp.where(logits == m1, eidx, E), axis=1, keepdims=True)
    oh1 = eidx == idx1
    logits2 = jnp.where(oh1, -jnp.inf, logits)
    m2 = jnp.max(logits2, axis=1, keepdims=True)
    idx2 = jnp.min(jnp.where(logits2 == m2, eidx, E), axis=1, keepdims=True)
    oh2 = eidx == idx2
    e2 = jnp.exp(m2 - m1)
    wt1 = 1.0 / (1.0 + e2)
    wt2 = 1.0 - wt1

    oh = (oh1 | oh2).astype(jnp.float32)
    r_i = lax.broadcasted_iota(jnp.int32, (T, T), 0)
    c_i = lax.broadcasted_iota(jnp.int32, (T, T), 1)
    ltri = (c_i < r_i).astype(jnp.bfloat16)
    cum_excl = lax.dot_general(ltri, oh.astype(jnp.bfloat16),
                               (((1,), (0,)), ((), ())),
                               preferred_element_type=jnp.float32)
    counts = jnp.sum(oh, axis=0, keepdims=True)
    counts_i = counts.astype(jnp.int32)
    padded = ((counts_i + BLK - 1) // BLK) * BLK
    e_r = lax.broadcasted_iota(jnp.int32, (E, E), 0)
    e_c = lax.broadcasted_iota(jnp.int32, (E, E), 1)
    ue = (e_r < e_c).astype(jnp.float32)
    off_f = lax.dot_general(padded.astype(jnp.float32), ue,
                            (((1,), (0,)), ((), ())),
                            preferred_element_type=jnp.float32)
    rank0 = jnp.sum(jnp.where(oh1, cum_excl, 0.0), axis=1, keepdims=True)
    rank1 = jnp.sum(jnp.where(oh2, cum_excl, 0.0), axis=1, keepdims=True)
    off0 = jnp.sum(jnp.where(oh1, off_f, 0.0), axis=1, keepdims=True)
    off1 = jnp.sum(jnp.where(oh2, off_f, 0.0), axis=1, keepdims=True)
    pos0_ref[...] = (rank0 + off0).astype(jnp.int32)
    pos1_ref[...] = (rank1 + off1).astype(jnp.int32)
    w0_ref[...] = jnp.broadcast_to(wt1, (T, 128))
    w1_ref[...] = jnp.broadcast_to(wt2, (T, 128))

    off_i = off_f.astype(jnp.int32)
    endblk = (off_i + padded) // BLK
    total = jnp.sum(padded)
    nact = total // BLK
    b_i = lax.broadcasted_iota(jnp.int32, (NB, E), 0)
    be = jnp.sum((jnp.broadcast_to(endblk, (NB, E)) <= b_i).astype(jnp.int32),
                 axis=1, keepdims=True)
    be_last = jnp.sum((endblk <= nact - 1).astype(jnp.int32))
    b_col = lax.broadcasted_iota(jnp.int32, (NB, 1), 0)
    be_ref[...] = jnp.where(b_col >= nact, be_last, be)
    na_ref[...] = jnp.full((1, 1), nact, jnp.int32)


_DISPATCH_OUT = [
    jax.ShapeDtypeStruct((T, 1), jnp.int32),
    jax.ShapeDtypeStruct((T, 1), jnp.int32),
    jax.ShapeDtypeStruct((T, 128), jnp.float32),
    jax.ShapeDtypeStruct((T, 128), jnp.float32),
    jax.ShapeDtypeStruct((NB, 1), jnp.int32),
    jax.ShapeDtypeStruct((1, 1), jnp.int32),
]


@functools.lru_cache(maxsize=1)
def _scatter_sc():
    mesh = plsc.VectorSubcoreMesh(core_axis_name="c", subcore_axis_name="s",
                                  num_cores=NC, num_subcores=NS)

    @functools.partial(
        pl.kernel,
        out_type=[jax.ShapeDtypeStruct((P, H), jnp.float32),
                  jax.ShapeDtypeStruct((P, 128), jnp.float32)],
        mesh=mesh,
        scratch_types=[pltpu.VMEM((TPW, H), jnp.float32),
                       pltpu.VMEM((TPW,), jnp.int32),
                       pltpu.VMEM((TPW, 128), jnp.float32),
                       pltpu.SemaphoreType.DMA,
                       pltpu.SemaphoreType.DMA],
    )
    def scatter(x_hbm, pos0_hbm, pos1_hbm, w0_hbm, w1_hbm,
                xg_hbm, ws_hbm, rows_v, idx_v, w_v, sem, sem2):
        wid = lax.axis_index("s") * NC + lax.axis_index("c")
        base = wid * TPW
        pltpu.sync_copy(x_hbm.at[pl.ds(base, TPW)], rows_v)
        for p_hbm, wk_hbm in ((pos0_hbm, w0_hbm), (pos1_hbm, w1_hbm)):
            pltpu.sync_copy(p_hbm.at[pl.ds(base, TPW)], idx_v)
            pltpu.sync_copy(wk_hbm.at[pl.ds(base, TPW)], w_v)
            c0 = pltpu.async_copy(rows_v, xg_hbm.at[idx_v], sem)
            c1 = pltpu.async_copy(w_v, ws_hbm.at[idx_v], sem2)
            c0.wait()
            c1.wait()

    return scatter


def _mlp_body(be_ref, na_ref, xg_ref, ws_ref, w1_ref, b1_ref, w2_ref, b2_ref,
              y_ref, h_scr):
    s = pl.program_id(0)

    @pl.when(s < na_ref[0])
    def _():
        xb = xg_ref[...].astype(jnp.bfloat16)
        h = lax.dot_general(xb, w1_ref[0].astype(jnp.bfloat16),
                            (((1,), (0,)), ((), ())),
                            preferred_element_type=jnp.float32)
        h = _gelu_exact(h + b1_ref[0]).astype(jnp.bfloat16)
        h_scr[lax.rem(s, 2)] = h

    @pl.when(jnp.logical_and(s >= 1, s - 1 < na_ref[0]))
    def _():
        y = lax.dot_general(h_scr[lax.rem(s - 1, 2)],
                            w2_ref[0].astype(jnp.bfloat16),
                            (((1,), (0,)), ((), ())),
                            preferred_element_type=jnp.float32)
        y = y + b2_ref[0]
        y_ref[...] = y * ws_ref[...][:, 0:1]


def _mlp_grid_spec():
    c0 = lambda s: jnp.minimum(s, NB - 1)
    c1 = lambda s: jnp.maximum(s - 1, 0)
    return pltpu.PrefetchScalarGridSpec(
        num_scalar_prefetch=2,
        grid=(NB + 1,),
        in_specs=[
            pl.BlockSpec((BLK, H), lambda s, be, na: (c0(s), 0)),
            pl.BlockSpec((BLK, 128), lambda s, be, na: (c1(s), 0)),
            pl.BlockSpec((1, H, F), lambda s, be, na: (be[c0(s)], 0, 0)),
            pl.BlockSpec((1, 1, F), lambda s, be, na: (be[c0(s)], 0, 0)),
            pl.BlockSpec((1, F, H), lambda s, be, na: (be[c1(s)], 0, 0)),
            pl.BlockSpec((1, 1, H), lambda s, be, na: (be[c1(s)], 0, 0)),
        ],
        out_specs=pl.BlockSpec((BLK, H), lambda s, be, na: (c1(s), 0)),
        scratch_shapes=[pltpu.VMEM((2, BLK, F), jnp.bfloat16)],
    )


@functools.lru_cache(maxsize=1)
def _combine_sc():
    mesh = plsc.VectorSubcoreMesh(core_axis_name="c", subcore_axis_name="s",
                                  num_cores=NC, num_subcores=NS)

    @functools.partial(
        pl.kernel,
        out_type=jax.ShapeDtypeStruct((T, H), jnp.float32),
        mesh=mesh,
        scratch_types=[pltpu.VMEM((SUB, H), jnp.float32),
                       pltpu.VMEM((SUB, H), jnp.float32),
                       pltpu.VMEM((SUB,), jnp.int32),
                       pltpu.VMEM((SUB,), jnp.int32),
                       pltpu.SemaphoreType.DMA,
                       pltpu.SemaphoreType.DMA],
    )
    def combine(y_hbm, pos0_hbm, pos1_hbm, out_hbm,
                buf0, buf1, i0, i1, s0, s1):
        wid = lax.axis_index("s") * NC + lax.axis_index("c")
        for sub in range(TPW // SUB):
            base = wid * TPW + sub * SUB
            pltpu.sync_copy(pos0_hbm.at[pl.ds(base, SUB)], i0)
            pltpu.sync_copy(pos1_hbm.at[pl.ds(base, SUB)], i1)
            c0 = pltpu.async_copy(y_hbm.at[i0], buf0, s0)
            c1 = pltpu.async_copy(y_hbm.at[i1], buf1, s1)
            c0.wait()
            c1.wait()

            @plsc.parallel_loop(0, SUB, unroll=2)
            def _addrow(r):
                for c in range(H // 16):
                    sl = pl.ds(c * 16, 16)
                    buf0[r, sl] = buf0[r, sl] + buf1[r, sl]

            pltpu.sync_copy(buf0, out_hbm.at[pl.ds(base, SUB)])

    return combine


def kernel(hidden_states, W_router, W1, b1, W2, b2):
    x2d = hidden_states.reshape(T, H)
    pos0, pos1, w0r, w1r, be, na = pl.pallas_call(
        _dispatch_body, out_shape=_DISPATCH_OUT)(x2d, W_router)
    pos0 = pos0.reshape(T)
    pos1 = pos1.reshape(T)

    xg, ws = _scatter_sc()(x2d, pos0, pos1, w0r, w1r)

    y = pl.pallas_call(
        _mlp_body,
        grid_spec=_mlp_grid_spec(),
        out_shape=jax.ShapeDtypeStruct((P, H), jnp.float32),
    )(be.reshape(NB), na.reshape(1), xg, ws,
      W1, b1.reshape(E, 1, F), W2, b2.reshape(E, 1, H))

    out = _combine_sc()(y, pos0, pos1)
    return out.reshape(hidden_states.shape)
